# baseline (speedup 1.0000x reference)
import jax
import jax.numpy as jnp
from jax.experimental import pallas as pl

_ROWS = 512


def _add_kernel(x_ref, e_ref, o_ref):
    o_ref[...] = x_ref[...] + e_ref[...]


def kernel(inputs, embeddings):
    b, s, d = inputs.shape
    x = inputs.reshape(b * s, d)
    nsb = s // _ROWS
    grid = (b * nsb,)
    out = pl.pallas_call(
        _add_kernel,
        grid=grid,
        in_specs=[
            pl.BlockSpec((_ROWS, d), lambda j: ((j % b) * nsb + j // b, 0)),
            pl.BlockSpec((_ROWS, d), lambda j: (j // b, 0)),
        ],
        out_specs=pl.BlockSpec((_ROWS, d), lambda j: ((j % b) * nsb + j // b, 0)),
        out_shape=jax.ShapeDtypeStruct((b * s, d), inputs.dtype),
    )(x, embeddings)
    return out.reshape(b, s, d)
